# baseline (speedup 1.0000x reference)
import functools

import jax
import jax.numpy as jnp
from jax import lax
from jax.experimental import pallas as pl
from jax.experimental.pallas import tpu as pltpu
from jax.experimental.pallas import tpu_sc as plsc

BATCH = 16384
EMBED_DIM = 64
NC = 2
NS = 16
NW = NC * NS
B_PER_W = BATCH // NW
CHUNK = 128
N_CHUNKS = B_PER_W // CHUNK
GROUP = 16
GROUPS_PER_CHUNK = CHUNK // GROUP


def _body(uid_hbm, aid_hbm, ut_hbm, at_hbm, out_hbm,
          uidx_v, aidx_v, upair_v, apair_v, ubuf_v, abuf_v, out_v, sem):
    wid = lax.axis_index("s") * NC + lax.axis_index("c")
    base = wid * B_PER_W

    pltpu.sync_copy(uid_hbm.at[pl.ds(base, B_PER_W)], uidx_v)
    pltpu.sync_copy(aid_hbm.at[pl.ds(base, B_PER_W)], aidx_v)

    for i in range(B_PER_W // 16):
        sl = pl.ds(i * 16, 16)
        upair_v[sl] = lax.shift_right_logical(uidx_v[sl], 1)
        apair_v[sl] = lax.shift_right_logical(aidx_v[sl], 1)

    lane = lax.iota(jnp.int32, 16)

    def chunk_step(c, carry):
        csl = pl.ds(c * CHUNK, CHUNK)
        cu = pltpu.async_copy(ut_hbm.at[upair_v.at[csl]], ubuf_v, sem)
        ca = pltpu.async_copy(at_hbm.at[apair_v.at[csl]], abuf_v, sem)
        cu.wait()
        ca.wait()

        def group_step(g, carry2):
            isl = pl.ds(c * CHUNK + g * GROUP, 16)
            ucol = (uidx_v[isl] & 1) * 64
            acol = (aidx_v[isl] & 1) * 64
            row = g * GROUP + lane
            acc = plsc.load_gather(ubuf_v, [row, ucol]) * plsc.load_gather(
                abuf_v, [row, acol])
            for d in range(1, EMBED_DIM):
                acc = acc + plsc.load_gather(ubuf_v, [row, ucol + d]) * (
                    plsc.load_gather(abuf_v, [row, acol + d]))
            out_v[pl.ds(c * CHUNK + g * GROUP, 16)] = 1.0 / (1.0 + jnp.exp(-acc))
            return carry2

        lax.fori_loop(0, GROUPS_PER_CHUNK, group_step, 0)
        return carry

    lax.fori_loop(0, N_CHUNKS, chunk_step, 0)

    pltpu.sync_copy(out_v, out_hbm.at[pl.ds(base, B_PER_W)])


@jax.jit
def _run(user_ids, anime_ids, user_table, anime_table):
    ut2 = jnp.reshape(user_table, (50000, 128))
    at2 = jnp.reshape(anime_table, (50000, 128))
    mesh = plsc.VectorSubcoreMesh(core_axis_name="c", subcore_axis_name="s")
    k = functools.partial(
        pl.kernel,
        mesh=mesh,
        compiler_params=pltpu.CompilerParams(
            needs_layout_passes=False, use_tc_tiling_on_sc=True),
        out_type=jax.ShapeDtypeStruct((BATCH,), jnp.float32),
        scratch_types=[
            pltpu.VMEM((B_PER_W,), jnp.int32),
            pltpu.VMEM((B_PER_W,), jnp.int32),
            pltpu.VMEM((B_PER_W,), jnp.int32),
            pltpu.VMEM((B_PER_W,), jnp.int32),
            pltpu.VMEM((CHUNK, 128), jnp.float32),
            pltpu.VMEM((CHUNK, 128), jnp.float32),
            pltpu.VMEM((B_PER_W,), jnp.float32),
            pltpu.SemaphoreType.DMA,
        ],
    )(_body)
    return k(user_ids, anime_ids, ut2, at2)


def kernel(user_ids, anime_ids, user_table, anime_table):
    return _run(jnp.asarray(user_ids, jnp.int32), jnp.asarray(anime_ids, jnp.int32),
                user_table, anime_table)
